# baseline (speedup 1.0000x reference)
import jax
import jax.numpy as jnp
from jax.experimental import pallas as pl
from jax.experimental.pallas import tpu as pltpu

NUM_EXPERTS = 64
D_MODEL = 1024
D_FF = 512


def _moe_body(x_ref, gw_ref, b_ref, w1_hbm, w2_hbm, w3_hbm, out_ref,
              w1_b, w2_b, w3_b, sem):
    xv = x_ref[...]

    logits = jnp.sum(gw_ref[...] * xv, axis=1, keepdims=True)
    scores = jax.nn.sigmoid(logits)
    biased = scores + b_ref[...]
    iota = jax.lax.broadcasted_iota(jnp.int32, (NUM_EXPERTS, 1), 0)
    m0 = jnp.max(biased)
    i0 = jnp.min(jnp.where(biased == m0, iota, NUM_EXPERTS))

    FC = D_FF // 2
    DC = D_MODEL // 2
    copies = []

    def issue_expert(k, e):
        for c in range(2):
            copies.append(pltpu.make_async_copy(
                w1_hbm.at[e, pl.ds(c * FC, FC)],
                w1_b.at[k, pl.ds(c * FC, FC)], sem.at[len(copies)]))
            copies[-1].start()
            copies.append(pltpu.make_async_copy(
                w3_hbm.at[e, pl.ds(c * FC, FC)],
                w3_b.at[k, pl.ds(c * FC, FC)], sem.at[len(copies)]))
            copies[-1].start()
        for c in range(2):
            copies.append(pltpu.make_async_copy(
                w2_hbm.at[e, pl.ds(c * DC, DC)],
                w2_b.at[k, pl.ds(c * DC, DC)], sem.at[len(copies)]))
            copies[-1].start()

    issue_expert(0, i0)
    masked = jnp.where(iota == i0, -jnp.inf, biased)
    m1 = jnp.max(masked)
    i1 = jnp.min(jnp.where(masked == m1, iota, NUM_EXPERTS))
    issue_expert(1, i1)
    s0 = jnp.sum(jnp.where(iota == i0, scores, 0.0))
    s1 = jnp.sum(jnp.where(iota == i1, scores, 0.0))
    inv = 1.0 / (s0 + s1 + 1e-20)

    def dotT(a, b):
        return jax.lax.dot_general(
            a, b, (((1,), (1,)), ((), ())), preferred_element_type=jnp.float32)

    def expert(k, base, scale):
        gs = []
        for c in range(2):
            copies[base + 2 * c].wait()
            h = dotT(xv, w1_b[k, pl.ds(c * FC, FC)])
            copies[base + 2 * c + 1].wait()
            u = dotT(xv, w3_b[k, pl.ds(c * FC, FC)])
            gs.append(jax.nn.silu(h) * u * scale)
        g = jnp.concatenate(gs, axis=1)
        os = []
        for c in range(2):
            copies[base + 4 + c].wait()
            os.append(dotT(g, w2_b[k, pl.ds(c * DC, DC)]))
        return jnp.concatenate(os, axis=1)

    o0 = expert(0, 0, s0 * inv)
    o1 = expert(1, 6, s1 * inv)
    out_ref[...] = o0 + o1


def kernel(x, gate_weight, bias, W1, W2, W3):
    x2 = x.astype(jnp.float32).reshape(1, D_MODEL)
    b2 = bias.reshape(NUM_EXPERTS, 1)
    out = pl.pallas_call(
        _moe_body,
        out_shape=jax.ShapeDtypeStruct((1, D_MODEL), jnp.float32),
        in_specs=[
            pl.BlockSpec(memory_space=pltpu.VMEM),
            pl.BlockSpec(memory_space=pltpu.VMEM),
            pl.BlockSpec(memory_space=pltpu.VMEM),
            pl.BlockSpec(memory_space=pl.ANY),
            pl.BlockSpec(memory_space=pl.ANY),
            pl.BlockSpec(memory_space=pl.ANY),
        ],
        out_specs=pl.BlockSpec(memory_space=pltpu.VMEM),
        scratch_shapes=[
            pltpu.VMEM((2, D_FF, D_MODEL), jnp.float32),
            pltpu.VMEM((2, D_MODEL, D_FF), jnp.float32),
            pltpu.VMEM((2, D_FF, D_MODEL), jnp.float32),
            pltpu.SemaphoreType.DMA((12,)),
        ],
    )(x2, gate_weight, b2, W1, W2, W3)
    return out.reshape(1, 1, 1, D_MODEL)


# speedup vs baseline: 1.1823x; 1.1823x over previous
import jax
import jax.numpy as jnp
from jax.experimental import pallas as pl
from jax.experimental.pallas import tpu as pltpu

NUM_EXPERTS = 64
D_MODEL = 1024
D_FF = 512


def _moe_body(x_ref, gw_ref, b_ref, w1_hbm, w2_hbm, w3_hbm, out_ref,
              w1_b, w2_b, w3_b, sem):
    xv = x_ref[...]

    logits = jax.lax.dot_general(
        xv, gw_ref[...], (((1,), (1,)), ((), ())),
        preferred_element_type=jnp.float32)
    scores = jax.nn.sigmoid(logits)
    biased = scores + b_ref[...]
    iota = jax.lax.broadcasted_iota(jnp.int32, (1, NUM_EXPERTS), 1)
    m0 = jnp.max(biased)
    i0 = jnp.min(jnp.where(biased == m0, iota, NUM_EXPERTS))

    FC = D_FF // 2
    DC = D_MODEL // 2
    copies = []

    def issue_expert(k, e):
        for c in range(2):
            copies.append(pltpu.make_async_copy(
                w1_hbm.at[e, pl.ds(c * FC, FC)],
                w1_b.at[k, pl.ds(c * FC, FC)], sem.at[len(copies)]))
            copies[-1].start()
            copies.append(pltpu.make_async_copy(
                w3_hbm.at[e, pl.ds(c * FC, FC)],
                w3_b.at[k, pl.ds(c * FC, FC)], sem.at[len(copies)]))
            copies[-1].start()
        for c in range(2):
            copies.append(pltpu.make_async_copy(
                w2_hbm.at[e, pl.ds(c * DC, DC)],
                w2_b.at[k, pl.ds(c * DC, DC)], sem.at[len(copies)]))
            copies[-1].start()

    issue_expert(0, i0)
    masked = jnp.where(iota == i0, -jnp.inf, biased)
    m1 = jnp.max(masked)
    i1 = jnp.min(jnp.where(masked == m1, iota, NUM_EXPERTS))
    issue_expert(1, i1)
    s0 = jnp.sum(jnp.where(iota == i0, scores, 0.0))
    s1 = jnp.sum(jnp.where(iota == i1, scores, 0.0))
    inv = 1.0 / (s0 + s1 + 1e-20)

    def dotT(a, b):
        return jax.lax.dot_general(
            a, b, (((1,), (1,)), ((), ())), preferred_element_type=jnp.float32)

    def expert(k, base, scale):
        gs = []
        for c in range(2):
            copies[base + 2 * c].wait()
            h = dotT(xv, w1_b[k, pl.ds(c * FC, FC)])
            copies[base + 2 * c + 1].wait()
            u = dotT(xv, w3_b[k, pl.ds(c * FC, FC)])
            gs.append(jax.nn.silu(h) * u * scale)
        g = jnp.concatenate(gs, axis=1)
        os = []
        for c in range(2):
            copies[base + 4 + c].wait()
            os.append(dotT(g, w2_b[k, pl.ds(c * DC, DC)]))
        return jnp.concatenate(os, axis=1)

    o0 = expert(0, 0, s0 * inv)
    o1 = expert(1, 6, s1 * inv)
    out_ref[...] = o0 + o1


def kernel(x, gate_weight, bias, W1, W2, W3):
    x2 = x.astype(jnp.float32).reshape(1, D_MODEL)
    b2 = bias.reshape(1, NUM_EXPERTS)
    out = pl.pallas_call(
        _moe_body,
        out_shape=jax.ShapeDtypeStruct((1, D_MODEL), jnp.float32),
        in_specs=[
            pl.BlockSpec(memory_space=pltpu.VMEM),
            pl.BlockSpec(memory_space=pltpu.VMEM),
            pl.BlockSpec(memory_space=pltpu.VMEM),
            pl.BlockSpec(memory_space=pl.ANY),
            pl.BlockSpec(memory_space=pl.ANY),
            pl.BlockSpec(memory_space=pl.ANY),
        ],
        out_specs=pl.BlockSpec(memory_space=pltpu.VMEM),
        scratch_shapes=[
            pltpu.VMEM((2, D_FF, D_MODEL), jnp.float32),
            pltpu.VMEM((2, D_MODEL, D_FF), jnp.float32),
            pltpu.VMEM((2, D_FF, D_MODEL), jnp.float32),
            pltpu.SemaphoreType.DMA((12,)),
        ],
    )(x2, gate_weight, b2, W1, W2, W3)
    return out.reshape(1, 1, 1, D_MODEL)
